# baseline (speedup 1.0000x reference)
import jax
import jax.numpy as jnp
from jax.experimental import pallas as pl

N_ROIS = 5000
FEAT = 12544
N_OUT = 32 + 124 + 3 + 7 + 2
M_BLK = 200


def _fused_heads_kernel(x_ref, w_ref, b_ref, o_ref):
    xb = x_ref[...].astype(jnp.bfloat16)
    acc = jnp.dot(xb, w_ref[...], preferred_element_type=jnp.float32)
    o_ref[...] = acc + b_ref[...]


def _fused_matmul(xf, w_all, b_all):
    grid = (N_ROIS // M_BLK,)
    return pl.pallas_call(
        _fused_heads_kernel,
        grid=grid,
        in_specs=[
            pl.BlockSpec((M_BLK, FEAT), lambda i: (i, 0)),
            pl.BlockSpec((FEAT, N_OUT), lambda i: (0, 0)),
            pl.BlockSpec((1, N_OUT), lambda i: (0, 0)),
        ],
        out_specs=pl.BlockSpec((M_BLK, N_OUT), lambda i: (i, 0)),
        out_shape=jax.ShapeDtypeStruct((N_ROIS, N_OUT), jnp.float32),
    )(xf, w_all, b_all)


def kernel(x, W_cls, b_cls, W_reg, b_reg, W_face, b_face, W_colour, b_colour, W_motion, b_motion):
    xf = x.reshape(x.shape[0], -1)
    w_all = jnp.concatenate([W_cls, W_reg, W_face, W_colour, W_motion], axis=0)
    w_all = w_all.T.astype(jnp.bfloat16)
    b_all = jnp.concatenate([b_cls, b_reg, b_face, b_colour, b_motion])[None, :]
    out = _fused_matmul(xf, w_all, b_all)
    n_cls = W_cls.shape[0]
    n_reg = W_reg.shape[0]
    n_face = W_face.shape[0]
    n_colour = W_colour.shape[0]
    o1 = n_cls
    o2 = o1 + n_reg
    o3 = o2 + n_face
    o4 = o3 + n_colour
    return (
        out[:, :o1],
        out[:, o1:o2],
        out[:, o2:o3],
        out[:, o3:o4],
        out[:, o4:],
    )


# speedup vs baseline: 5.1637x; 5.1637x over previous
import jax
import jax.numpy as jnp
from jax.experimental import pallas as pl

N_ROIS = 5000
IN_CH = 256
ROI = 7
N_OUT = 32 + 124 + 3 + 7 + 2
M_BLK = 200


def _fused_heads_kernel(x_ref, w_ref, b_ref, o_ref):
    acc = b_ref[...].astype(jnp.float32)
    for i in range(ROI):
        for j in range(ROI):
            xs = x_ref[i, j].astype(jnp.bfloat16)
            acc = acc + jnp.dot(xs, w_ref[i, j],
                                preferred_element_type=jnp.float32)
    o_ref[...] = acc


def _fused_matmul(xt, w4, b_all):
    grid = (N_ROIS // M_BLK,)
    return pl.pallas_call(
        _fused_heads_kernel,
        grid=grid,
        in_specs=[
            pl.BlockSpec((ROI, ROI, M_BLK, IN_CH), lambda i: (0, 0, i, 0)),
            pl.BlockSpec((ROI, ROI, IN_CH, N_OUT), lambda i: (0, 0, 0, 0)),
            pl.BlockSpec((1, N_OUT), lambda i: (0, 0)),
        ],
        out_specs=pl.BlockSpec((M_BLK, N_OUT), lambda i: (i, 0)),
        out_shape=jax.ShapeDtypeStruct((N_ROIS, N_OUT), jnp.float32),
    )(xt, w4, b_all)


def kernel(x, W_cls, b_cls, W_reg, b_reg, W_face, b_face, W_colour, b_colour, W_motion, b_motion):
    xt = jnp.transpose(x, (2, 3, 0, 1))
    w_all = jnp.concatenate([W_cls, W_reg, W_face, W_colour, W_motion], axis=0)
    w4 = w_all.reshape(N_OUT, IN_CH, ROI, ROI).transpose(2, 3, 1, 0)
    w4 = w4.astype(jnp.bfloat16)
    b_all = jnp.concatenate([b_cls, b_reg, b_face, b_colour, b_motion])[None, :]
    out = _fused_matmul(xt, w4, b_all)
    n_cls = W_cls.shape[0]
    n_reg = W_reg.shape[0]
    n_face = W_face.shape[0]
    n_colour = W_colour.shape[0]
    o1 = n_cls
    o2 = o1 + n_reg
    o3 = o2 + n_face
    o4 = o3 + n_colour
    return (
        out[:, :o1],
        out[:, o1:o2],
        out[:, o2:o3],
        out[:, o3:o4],
        out[:, o4:],
    )
